# baseline (speedup 1.0000x reference)
import jax
import jax.numpy as jnp
from jax.experimental import pallas as pl
from jax.experimental.pallas import tpu as pltpu

B, Cin, H, P, D = 16, 3, 384, 16, 768
G = H // P
N = G * G


def _gelu_exact(x):
    return x * 0.5 * (1.0 + jax.lax.erf(x * 0.7071067811865476))


def _ln_rows(x, g, b, eps=1e-5):
    m = jnp.mean(x, axis=-1, keepdims=True)
    v = jnp.mean((x - m) ** 2, axis=-1, keepdims=True)
    return (x - m) / jnp.sqrt(v + eps) * g + b


def _bdot(a, w):
    return jnp.dot(a.astype(jnp.bfloat16), w,
                   preferred_element_type=jnp.float32)


def _f32dot(a, w):
    return jnp.dot(a, w, preferred_element_type=jnp.float32)


def _fused_body(xp1_ref, xp2_ref, pos_ref, cls_t_ref,
                WpT_ref, b_patch_ref,
                lnl_g_ref, lnl_b_ref, W_local_ref, b_local_ref,
                lnc_g_ref, lnc_b_ref, W_cls_ref, b_cls_ref,
                Wo1_ref, bo1_ref,
                Wo2_ref, bo2_ref, wd_ref, bd_ref,
                lnm_g_ref, lnm_b_ref, W_a_ref, W_g_ref, W_out_ref,
                out_ref,
                zc_ref, cls_o_ref,
                wpt_bf, wlocal_bf, wo1t_bf, wo2_bf, wag_bf, wout_bf):
    b = pl.program_id(0)
    bf16 = jnp.bfloat16

    @pl.when(b == 0)
    def _prep():
        wpt_bf[...] = WpT_ref[...].astype(bf16)
        wlocal_bf[...] = W_local_ref[...].astype(bf16)
        wo1t_bf[...] = Wo1_ref[: D // 2].astype(bf16)
        wo2_bf[...] = Wo2_ref[...].astype(bf16)
        wag_bf[:, : 2 * D] = W_a_ref[...].astype(bf16)
        wag_bf[:, 2 * D:] = W_g_ref[...].astype(bf16)
        wout_bf[...] = W_out_ref[...].astype(bf16)

        cls_t = cls_t_ref[...]
        hn_c = _ln_rows(cls_t, lnc_g_ref[...], lnc_b_ref[...])
        cls_f = _gelu_exact(_f32dot(hn_c, W_cls_ref[...]) + b_cls_ref[...])
        zc_ref[...] = _f32dot(cls_f, Wo1_ref[D // 2:]) + bo1_ref[...]
        hn_cm = _ln_rows(cls_t, lnm_g_ref[...], lnm_b_ref[...])
        a_c = _f32dot(hn_cm, W_a_ref[...])
        g_c = _f32dot(hn_cm, W_g_ref[...])
        cls_o_ref[...] = cls_t + _f32dot(a_c * jax.nn.silu(g_c),
                                         W_out_ref[...])

    xb = jnp.where(b < B // 2, xp1_ref[0], xp2_ref[0])
    out_ref[0, 0:1, :] = cls_o_ref[...]
    HN = N // 2
    for h in range(2):
        t = (_bdot(xb[h * HN:(h + 1) * HN], wpt_bf[...])
             + b_patch_ref[...] + pos_ref[h * HN:(h + 1) * HN])

        hn_l = _ln_rows(t, lnl_g_ref[...], lnl_b_ref[...])
        lx = _gelu_exact(_bdot(hn_l, wlocal_bf[...]) + b_local_ref[...])

        z1 = _gelu_exact(_bdot(lx, wo1t_bf[...]) + zc_ref[...])
        z2 = _gelu_exact(_bdot(z1, wo2_bf[...]) + bo2_ref[...])
        logit = _bdot(z2, wd_ref[...]) + bd_ref[...]
        p = jax.nn.sigmoid(logit)

        m = t * p

        hn = _ln_rows(m, lnm_g_ref[...], lnm_b_ref[...])
        ag = _bdot(hn, wag_bf[...])
        a = ag[:, : 2 * D]
        g = ag[:, 2 * D:]
        mix = _bdot(a * jax.nn.silu(g), wout_bf[...])
        out_ref[0, 1 + h * HN: 1 + (h + 1) * HN, :] = m + mix


def kernel(x, W_patch, b_patch, cls_token, pos_embed,
           ln_local_g, ln_local_b, W_local, b_local,
           ln_cls_g, ln_cls_b, W_cls, b_cls,
           Wo1, bo1, Wo2, bo2, Wo3, bo3,
           ln_m_g, ln_m_b, W_a, W_g, W_out):
    f32 = jnp.float32
    bf16 = jnp.bfloat16

    def _xp(xh):
        return (xh.reshape(B // 2, Cin, G, P, G, P)
                  .transpose(0, 2, 4, 1, 3, 5)
                  .reshape(B // 2, N, Cin * P * P))

    cls_t = (cls_token[0] + pos_embed[0, 0:1]).astype(f32)
    pos_tok = pos_embed[0, 1:, :]

    WpT = W_patch.T
    wd = (Wo3[:, 0:1] - Wo3[:, 1:2]).astype(bf16)
    bd_arr = (bo3[0] - bo3[1]).reshape(1, 1)

    parts = [_xp(x[i * (B // 2):(i + 1) * (B // 2)]) for i in range(2)]

    row = lambda v: v.reshape(1, -1)
    full = lambda shape: pl.BlockSpec(shape, lambda b: (0,) * len(shape))

    H2 = B // 2
    out = pl.pallas_call(
        _fused_body,
        grid=(B,),
        in_specs=[
            pl.BlockSpec((1, N, Cin * P * P),
                         lambda b: (jnp.minimum(b, H2 - 1), 0, 0)),
            pl.BlockSpec((1, N, Cin * P * P),
                         lambda b: (jnp.maximum(b - H2, 0), 0, 0)),
            full((N, D)),
            full((1, D)),
            full((Cin * P * P, D)),
            full((1, D)),
            full((1, D)), full((1, D)),
            full((D, D // 2)), full((1, D // 2)),
            full((1, D)), full((1, D)),
            full((D, D // 2)), full((1, D // 2)),
            full((D, D // 2)),
            full((1, D // 2)),
            full((D // 2, D // 4)), full((1, D // 4)),
            full((D // 4, 1)),
            full((1, 1)),
            full((1, D)), full((1, D)),
            full((D, 2 * D)), full((D, 2 * D)),
            full((2 * D, D)),
        ],
        out_specs=pl.BlockSpec((1, N + 1, D), lambda b: (b, 0, 0)),
        out_shape=jax.ShapeDtypeStruct((B, N + 1, D), f32),
        scratch_shapes=[
            pltpu.VMEM((1, D // 2), f32),
            pltpu.VMEM((1, D), f32),
            pltpu.VMEM((Cin * P * P, D), bf16),
            pltpu.VMEM((D, D // 2), bf16),
            pltpu.VMEM((D // 2, D // 2), bf16),
            pltpu.VMEM((D // 2, D // 4), bf16),
            pltpu.VMEM((D, 4 * D), bf16),
            pltpu.VMEM((2 * D, D), bf16),
        ],
    )(
        parts[0], parts[1], pos_tok, cls_t,
        WpT, row(b_patch),
        row(ln_local_g), row(ln_local_b), W_local, row(b_local),
        row(ln_cls_g), row(ln_cls_b), W_cls, row(b_cls),
        Wo1, row(bo1),
        Wo2, row(bo2), wd, bd_arr,
        row(ln_m_g), row(ln_m_b), W_a, W_g, W_out,
    )
    return out


# speedup vs baseline: 1.0129x; 1.0129x over previous
import jax
import jax.numpy as jnp
from jax.experimental import pallas as pl
from jax.experimental.pallas import tpu as pltpu

B, Cin, H, P, D = 16, 3, 384, 16, 768
G = H // P
N = G * G


def _gelu_exact(x):
    return x * 0.5 * (1.0 + jax.lax.erf(x * 0.7071067811865476))


def _ln_rows(x, g, b, eps=1e-5):
    m = jnp.mean(x, axis=-1, keepdims=True)
    v = jnp.mean((x - m) ** 2, axis=-1, keepdims=True)
    return (x - m) / jnp.sqrt(v + eps) * g + b


def _bdot(a, w):
    return jnp.dot(a.astype(jnp.bfloat16), w,
                   preferred_element_type=jnp.float32)


def _f32dot(a, w):
    return jnp.dot(a, w, preferred_element_type=jnp.float32)


def _fused_body(xp1_ref, xp2_ref, pos_ref, cls_t_ref,
                WpT_ref, b_patch_ref,
                lnl_g_ref, lnl_b_ref, W_local_ref, b_local_ref,
                lnc_g_ref, lnc_b_ref, W_cls_ref, b_cls_ref,
                Wo1_ref, bo1_ref,
                Wo2_ref, bo2_ref, wd_ref, bd_ref,
                lnm_g_ref, lnm_b_ref, W_a_ref, W_g_ref, W_out_ref,
                out_ref,
                zc_ref, cls_o_ref,
                wpt_bf, wlocal_bf, wo1t_bf, wo2_bf, wag_bf, wout_bf):
    b = pl.program_id(0)
    bf16 = jnp.bfloat16

    @pl.when(b == 0)
    def _prep():
        wpt_bf[...] = WpT_ref[...].astype(bf16)
        wlocal_bf[...] = W_local_ref[...].astype(bf16)
        wo1t_bf[...] = Wo1_ref[: D // 2].astype(bf16)
        wo2_bf[...] = Wo2_ref[...].astype(bf16)
        wag_bf[:, : 2 * D] = W_a_ref[...].astype(bf16)
        wag_bf[:, 2 * D:] = W_g_ref[...].astype(bf16)
        wout_bf[...] = W_out_ref[...].astype(bf16)

        cls_t = cls_t_ref[...]
        hn_c = _ln_rows(cls_t, lnc_g_ref[...], lnc_b_ref[...])
        cls_f = _gelu_exact(_f32dot(hn_c, W_cls_ref[...]) + b_cls_ref[...])
        zc_ref[...] = _f32dot(cls_f, Wo1_ref[D // 2:]) + bo1_ref[...]
        hn_cm = _ln_rows(cls_t, lnm_g_ref[...], lnm_b_ref[...])
        a_c = _f32dot(hn_cm, W_a_ref[...])
        g_c = _f32dot(hn_cm, W_g_ref[...])
        cls_o_ref[...] = cls_t + _f32dot(a_c * jax.nn.silu(g_c),
                                         W_out_ref[...])

    xb = jnp.where(b < B // 2, xp1_ref[0], xp2_ref[0])
    t = _bdot(xb, wpt_bf[...]) + b_patch_ref[...] + pos_ref[...]

    hn_l = _ln_rows(t, lnl_g_ref[...], lnl_b_ref[...])
    lx = _gelu_exact(_bdot(hn_l, wlocal_bf[...]) + b_local_ref[...])

    z1 = _gelu_exact(_bdot(lx, wo1t_bf[...]) + zc_ref[...])
    z2 = _gelu_exact(_bdot(z1, wo2_bf[...]) + bo2_ref[...])
    logit = _bdot(z2, wd_ref[...]) + bd_ref[...]
    p = jax.nn.sigmoid(logit)

    m = t * p

    hn = _ln_rows(m, lnm_g_ref[...], lnm_b_ref[...])
    ag = _bdot(hn, wag_bf[...])
    a = ag[:, : 2 * D]
    g = ag[:, 2 * D:]
    mix = _bdot(a * jax.nn.silu(g), wout_bf[...])
    out_ref[0, 0:1, :] = cls_o_ref[...]
    out_ref[0, 1:, :] = m + mix


def kernel(x, W_patch, b_patch, cls_token, pos_embed,
           ln_local_g, ln_local_b, W_local, b_local,
           ln_cls_g, ln_cls_b, W_cls, b_cls,
           Wo1, bo1, Wo2, bo2, Wo3, bo3,
           ln_m_g, ln_m_b, W_a, W_g, W_out):
    f32 = jnp.float32
    bf16 = jnp.bfloat16

    def _xp(xh):
        return (xh.reshape(B // 2, Cin, G, P, G, P)
                  .transpose(0, 2, 4, 1, 3, 5)
                  .reshape(B // 2, N, Cin * P * P))

    cls_t = (cls_token[0] + pos_embed[0, 0:1]).astype(f32)
    pos_tok = pos_embed[0, 1:, :]

    WpT = W_patch.T
    wd = (Wo3[:, 0:1] - Wo3[:, 1:2]).astype(bf16)
    bd_arr = (bo3[0] - bo3[1]).reshape(1, 1)

    parts = [_xp(x[i * (B // 2):(i + 1) * (B // 2)]) for i in range(2)]

    row = lambda v: v.reshape(1, -1)
    full = lambda shape: pl.BlockSpec(shape, lambda b: (0,) * len(shape))

    H2 = B // 2
    out = pl.pallas_call(
        _fused_body,
        grid=(B,),
        in_specs=[
            pl.BlockSpec((1, N, Cin * P * P),
                         lambda b: (jnp.minimum(b, H2 - 1), 0, 0)),
            pl.BlockSpec((1, N, Cin * P * P),
                         lambda b: (jnp.maximum(b - H2, 0), 0, 0)),
            full((N, D)),
            full((1, D)),
            full((Cin * P * P, D)),
            full((1, D)),
            full((1, D)), full((1, D)),
            full((D, D // 2)), full((1, D // 2)),
            full((1, D)), full((1, D)),
            full((D, D // 2)), full((1, D // 2)),
            full((D, D // 2)),
            full((1, D // 2)),
            full((D // 2, D // 4)), full((1, D // 4)),
            full((D // 4, 1)),
            full((1, 1)),
            full((1, D)), full((1, D)),
            full((D, 2 * D)), full((D, 2 * D)),
            full((2 * D, D)),
        ],
        out_specs=pl.BlockSpec((1, N + 1, D), lambda b: (b, 0, 0)),
        out_shape=jax.ShapeDtypeStruct((B, N + 1, D), f32),
        scratch_shapes=[
            pltpu.VMEM((1, D // 2), f32),
            pltpu.VMEM((1, D), f32),
            pltpu.VMEM((Cin * P * P, D), bf16),
            pltpu.VMEM((D, D // 2), bf16),
            pltpu.VMEM((D // 2, D // 2), bf16),
            pltpu.VMEM((D // 2, D // 4), bf16),
            pltpu.VMEM((D, 4 * D), bf16),
            pltpu.VMEM((2 * D, D), bf16),
        ],
    )(
        parts[0], parts[1], pos_tok, cls_t,
        WpT, row(b_patch),
        row(ln_local_g), row(ln_local_b), W_local, row(b_local),
        row(ln_cls_g), row(ln_cls_b), W_cls, row(b_cls),
        Wo1, row(bo1),
        Wo2, row(bo2), wd, bd_arr,
        row(ln_m_g), row(ln_m_b), W_a, W_g, W_out,
    )
    return out
